# baseline (speedup 1.0000x reference)
import functools

import jax
import jax.numpy as jnp
from jax import lax
from jax.experimental import pallas as pl
from jax.experimental.pallas import tpu as pltpu
from jax.experimental.pallas import tpu_sc as plsc

F32 = jnp.float32
BF16 = jnp.bfloat16
I32 = jnp.int32
NEG_INF = float("-inf")
MTILE = 256


def _gelu(x):
    return 0.5 * x * (1.0 + lax.erf(x * (2.0 ** -0.5)))


def _gate_univ_body(E, tokens_ref, task_ids_ref, task_embed_ref, gate_w_ref,
                    gate_b_ref, uw1_ref, ub1_ref, uw2_ref, ub2_ref,
                    logits_ref, pos1_ref, pos2_ref, g1_ref, g2_ref,
                    teid_ref, tu_ref, xbf_s, om_s):
    f = pl.program_id(0)
    N = tokens_ref.shape[1]
    D = tokens_ref.shape[2]

    @pl.when(f == 0)
    def _gating():
        x = tokens_ref[0]
        xbf_s[...] = x.astype(BF16)
        tid = task_ids_ref[0]
        te = task_embed_ref[...]
        tlog = jnp.dot(te, gate_w_ref[D:, :])
        tio = lax.broadcasted_iota(I32, tlog.shape, 0)
        tsel = jnp.sum(jnp.where(tio == tid, tlog, 0.0), axis=0, keepdims=True)
        logits = (jnp.dot(x, gate_w_ref[:D, :])
                  + tsel + gate_b_ref[...][None, :])
        logits_ref[0] = logits
        io8 = lax.broadcasted_iota(I32, (N, E), 1)
        v1 = jnp.max(logits, axis=1, keepdims=True)
        i1 = jnp.min(jnp.where(logits == v1, io8, E), axis=1, keepdims=True)
        is1 = io8 == i1
        neg = jnp.where(is1, NEG_INF, logits)
        v2 = jnp.max(neg, axis=1, keepdims=True)
        i2 = jnp.min(jnp.where(neg == v2, io8, E), axis=1, keepdims=True)
        is2 = io8 == i2
        r = jnp.exp(v2 - v1)
        g1 = 1.0 / (1.0 + r)
        g1_ref[...] = g1
        g2_ref[...] = r * g1
        om_s[...] = 1.0 - g1

        selF = jnp.where(is1 | is2, 1.0, 0.0)
        c = selF
        sh = 1
        while sh < N:
            shifted = jnp.concatenate(
                [jnp.zeros((sh, E), F32), c[:N - sh]], axis=0)
            c = c + shifted
            sh *= 2
        c_excl = c - selF
        total = c[N - 1:N, :]
        padded = jnp.floor((total + (MTILE - 1)) * (1.0 / MTILE)) * MTILE
        offinc = padded
        sh = 1
        while sh < E:
            shifted = jnp.concatenate(
                [jnp.zeros((1, sh), F32), offinc[:, :E - sh]], axis=1)
            offinc = offinc + shifted
            sh *= 2
        off = offinc - padded
        offend = offinc
        pos1 = jnp.sum(jnp.where(is1, off + c_excl, 0.0),
                       axis=1, keepdims=True)
        pos2 = jnp.sum(jnp.where(is2, off + c_excl, 0.0),
                       axis=1, keepdims=True)
        pos1_ref[...] = pos1.astype(I32)
        pos2_ref[...] = pos2.astype(I32)
        tb = (lax.broadcasted_iota(I32, (1, 32), 1) * MTILE).astype(F32)
        acc = jnp.zeros((1, 32), F32)
        for e2 in range(E):
            acc = acc + jnp.where(tb >= offend[:, e2:e2 + 1], 1.0, 0.0)
        teid_ref[...] = acc.astype(I32)

    xb = xbf_s[...]
    h = jnp.dot(xb, uw1_ref[...].astype(BF16), preferred_element_type=F32)
    h = _gelu(h + ub1_ref[0, 0][None, :])
    y = jnp.dot(h.astype(BF16), uw2_ref[...].astype(BF16),
                preferred_element_type=F32)
    y = jnp.where(f == 0, y + ub2_ref[0, 0][None, :], y)
    val = om_s[...] * y

    @pl.when(f == 0)
    def _init():
        tu_ref[0] = val

    @pl.when(f != 0)
    def _add():
        tu_ref[0] += val


def _scatter_rows_body(N, P, pos1_hbm, pos2_hbm, g1_hbm, g2_hbm,
                       row_token_hbm, row_gate_hbm,
                       p1_v, p2_v, g1buf_v, g2buf_v, tok_v, zi_v, zf_v, sem):
    cid = lax.axis_index("c")
    sid = lax.axis_index("s")
    TOK = N // 16
    tok0 = sid * TOK
    io16 = lax.broadcasted_iota(I32, (16,), 0)
    ZC = P // 16

    @pl.when(cid == 0)
    def _zero():
        for k in range(ZC // 16):
            zi_v[pl.ds(k * 16, 16)] = jnp.zeros((16,), I32)
            zf_v[pl.ds(k * 16, 16)] = jnp.zeros((16,), F32)
        pltpu.sync_copy(zi_v, row_token_hbm.at[pl.ds(sid * ZC, ZC)])
        pltpu.sync_copy(zf_v, row_gate_hbm.at[pl.ds(sid * ZC, ZC)])

    plsc.subcore_barrier()

    @pl.when(cid == 0)
    def _scatter():
        pltpu.sync_copy(pos1_hbm.at[pl.ds(tok0, TOK)], p1_v)
        pltpu.sync_copy(pos2_hbm.at[pl.ds(tok0, TOK)], p2_v)
        pltpu.sync_copy(g1_hbm.at[pl.ds(tok0, TOK)], g1buf_v)
        pltpu.sync_copy(g2_hbm.at[pl.ds(tok0, TOK)], g2buf_v)
        for k in range(TOK // 16):
            tok_v[pl.ds(k * 16, 16)] = tok0 + k * 16 + io16
        pltpu.async_copy(tok_v, row_token_hbm.at[p1_v], sem).wait()
        pltpu.async_copy(g1buf_v, row_gate_hbm.at[p1_v], sem).wait()
        pltpu.async_copy(tok_v, row_token_hbm.at[p2_v], sem).wait()
        pltpu.async_copy(g2buf_v, row_gate_hbm.at[p2_v], sem).wait()


def _gather_body(P, D, row_token_hbm, tokens_hbm, xs_hbm, idx_v, rows_v, sem):
    cid = lax.axis_index("c")
    sid = lax.axis_index("s")
    wid = sid * 2 + cid
    ROWS = P // 32
    CH = 32
    for ch in range(ROWS // CH):
        p0 = wid * ROWS + ch * CH
        pltpu.sync_copy(row_token_hbm.at[pl.ds(p0, CH)], idx_v)
        pltpu.async_copy(tokens_hbm.at[idx_v], rows_v, sem).wait()
        pltpu.sync_copy(rows_v, xs_hbm.at[pl.ds(p0, CH)])


def _grouped_body(eid_sref, x_ref, gate_ref, w1_ref, b1_ref, w2_ref, b2_ref,
                  y_ref):
    t = pl.program_id(0)
    f = pl.program_id(1)
    eid = eid_sref[t]

    @pl.when(eid < 8)
    def _compute():
        xb = x_ref[...].astype(BF16)
        h = jnp.dot(xb, w1_ref[0].astype(BF16), preferred_element_type=F32)
        h = _gelu(h + b1_ref[0, 0][None, :])
        y = jnp.dot(h.astype(BF16), w2_ref[0].astype(BF16),
                    preferred_element_type=F32)
        y = jnp.where(f == 0, y + b2_ref[0, 0][None, :], y)
        val = gate_ref[...] * y

        @pl.when(f == 0)
        def _init():
            y_ref[...] = val

        @pl.when(f != 0)
        def _add():
            y_ref[...] += val


def _combine_body(N, D, y_hbm, tu_hbm, pos1_hbm, pos2_hbm, out_hbm,
                  i1_v, i2_v, b1_v, b2_v, b3_v, sem):
    cid = lax.axis_index("c")
    sid = lax.axis_index("s")
    wid = sid * 2 + cid
    TOK = N // 32
    CH = 32
    for ch in range(TOK // CH):
        t0 = wid * TOK + ch * CH
        pltpu.sync_copy(pos1_hbm.at[pl.ds(t0, CH)], i1_v)
        pltpu.sync_copy(pos2_hbm.at[pl.ds(t0, CH)], i2_v)
        pltpu.async_copy(y_hbm.at[i1_v], b1_v, sem).wait()
        pltpu.async_copy(y_hbm.at[i2_v], b2_v, sem).wait()
        pltpu.sync_copy(tu_hbm.at[pl.ds(t0, CH)], b3_v)

        def _row(r, carry):
            for cc in range(D // 16):
                sl = pl.ds(cc * 16, 16)
                b3_v[r, sl] = b3_v[r, sl] + b1_v[r, sl] + b2_v[r, sl]
            return carry

        lax.fori_loop(0, CH, _row, 0)
        pltpu.sync_copy(b3_v, out_hbm.at[pl.ds(t0, CH)])


def _moe_sparse_core(tokens, task_ids, task_embed, gate_w, gate_b,
                     w1, b1, w2, b2, uw1, ub1, uw2, ub2):
    B, N, D = tokens.shape
    E = gate_w.shape[1]
    T = task_embed.shape[0]
    F = w1.shape[2]
    FC = 1024
    NF = F // FC
    NTILES = (2 * N) // MTILE + E
    P = NTILES * MTILE

    body1 = functools.partial(_gate_univ_body, E)
    logits, pos1, pos2, g1, g2, teid, tu = pl.pallas_call(
        body1,
        grid=(NF,),
        in_specs=[
            pl.BlockSpec((1, N, D), lambda f: (0, 0, 0)),
            pl.BlockSpec(memory_space=pltpu.SMEM),
            pl.BlockSpec((T, D), lambda f: (0, 0)),
            pl.BlockSpec((2 * D, E), lambda f: (0, 0)),
            pl.BlockSpec((E,), lambda f: (0,)),
            pl.BlockSpec((D, FC), lambda f: (0, f)),
            pl.BlockSpec((1, 1, FC), lambda f: (0, 0, f)),
            pl.BlockSpec((FC, D), lambda f: (f, 0)),
            pl.BlockSpec((1, 1, D), lambda f: (0, 0, 0)),
        ],
        out_specs=[
            pl.BlockSpec((1, N, E), lambda f: (0, 0, 0)),
            pl.BlockSpec((N, 1), lambda f: (0, 0)),
            pl.BlockSpec((N, 1), lambda f: (0, 0)),
            pl.BlockSpec((N, 1), lambda f: (0, 0)),
            pl.BlockSpec((N, 1), lambda f: (0, 0)),
            pl.BlockSpec((1, 32), lambda f: (0, 0)),
            pl.BlockSpec((1, N, D), lambda f: (0, 0, 0)),
        ],
        out_shape=[
            jax.ShapeDtypeStruct((B, N, E), F32),
            jax.ShapeDtypeStruct((N, 1), I32),
            jax.ShapeDtypeStruct((N, 1), I32),
            jax.ShapeDtypeStruct((N, 1), F32),
            jax.ShapeDtypeStruct((N, 1), F32),
            jax.ShapeDtypeStruct((1, 32), I32),
            jax.ShapeDtypeStruct((B, N, D), F32),
        ],
        scratch_shapes=[
            pltpu.VMEM((N, D), BF16),
            pltpu.VMEM((N, 1), F32),
        ],
    )(tokens, task_ids, task_embed, gate_w, gate_b,
      uw1, ub1.reshape(1, 1, F), uw2, ub2.reshape(1, 1, D))

    mesh = plsc.VectorSubcoreMesh(core_axis_name="c", subcore_axis_name="s")
    TOK = N // 16
    tile_eid = teid.reshape(32)

    scat = functools.partial(
        pl.kernel,
        mesh=mesh,
        out_type=[
            jax.ShapeDtypeStruct((P,), I32),
            jax.ShapeDtypeStruct((P,), F32),
        ],
        scratch_types=[
            pltpu.VMEM((TOK,), I32),
            pltpu.VMEM((TOK,), I32),
            pltpu.VMEM((TOK,), F32),
            pltpu.VMEM((TOK,), F32),
            pltpu.VMEM((TOK,), I32),
            pltpu.VMEM((P // 16,), I32),
            pltpu.VMEM((P // 16,), F32),
            pltpu.SemaphoreType.DMA,
        ],
    )(functools.partial(_scatter_rows_body, N, P))
    row_token, row_gate = scat(
        pos1.reshape(N), pos2.reshape(N), g1.reshape(N), g2.reshape(N))

    gath = functools.partial(
        pl.kernel,
        mesh=mesh,
        out_type=[jax.ShapeDtypeStruct((P, D), F32)],
        scratch_types=[
            pltpu.VMEM((32,), I32),
            pltpu.VMEM((32, D), F32),
            pltpu.SemaphoreType.DMA,
        ],
    )(functools.partial(_gather_body, P, D))
    (x_sorted,) = gath(row_token, tokens.reshape(N, D))

    grid_spec = pltpu.PrefetchScalarGridSpec(
        num_scalar_prefetch=1,
        grid=(NTILES, NF),
        in_specs=[
            pl.BlockSpec((MTILE, D), lambda t, f, eid: (t, 0)),
            pl.BlockSpec((MTILE, 1), lambda t, f, eid: (t, 0)),
            pl.BlockSpec((1, D, FC), lambda t, f, eid: (jnp.minimum(eid[t], 7), 0, f)),
            pl.BlockSpec((1, 1, FC), lambda t, f, eid: (jnp.minimum(eid[t], 7), 0, f)),
            pl.BlockSpec((1, FC, D), lambda t, f, eid: (jnp.minimum(eid[t], 7), f, 0)),
            pl.BlockSpec((1, 1, D), lambda t, f, eid: (jnp.minimum(eid[t], 7), 0, 0)),
        ],
        out_specs=pl.BlockSpec((MTILE, D), lambda t, f, eid: (t, 0)),
    )
    y = pl.pallas_call(
        _grouped_body,
        grid_spec=grid_spec,
        out_shape=jax.ShapeDtypeStruct((P, D), F32),
    )(tile_eid, x_sorted, row_gate.reshape(P, 1),
      w1, b1.reshape(E, 1, F), w2, b2.reshape(E, 1, D))

    comb = functools.partial(
        pl.kernel,
        mesh=mesh,
        out_type=[jax.ShapeDtypeStruct((N, D), F32)],
        scratch_types=[
            pltpu.VMEM((32,), I32),
            pltpu.VMEM((32,), I32),
            pltpu.VMEM((32, D), F32),
            pltpu.VMEM((32, D), F32),
            pltpu.VMEM((32, D), F32),
            pltpu.SemaphoreType.DMA,
        ],
    )(functools.partial(_combine_body, N, D))
    (t_out,) = comb(y, tu.reshape(N, D), pos1.reshape(N), pos2.reshape(N))

    return (t_out.reshape(B, N, D), logits, pos1, pos2, g1, g2, tile_eid,
            row_token, row_gate, x_sorted, y, tu)


def kernel(tokens, task_ids, task_embed, gate_w, gate_b,
           w1, b1, w2, b2, uw1, ub1, uw2, ub2):
    t_out, logits = _moe_sparse_core(
        tokens, task_ids, task_embed, gate_w, gate_b,
        w1, b1, w2, b2, uw1, ub1, uw2, ub2)[:2]
    return t_out, logits


# speedup vs baseline: 1.0432x; 1.0432x over previous
import functools

import jax
import jax.numpy as jnp
from jax import lax
from jax.experimental import pallas as pl
from jax.experimental.pallas import tpu as pltpu
from jax.experimental.pallas import tpu_sc as plsc

F32 = jnp.float32
BF16 = jnp.bfloat16
I32 = jnp.int32
NEG_INF = float("-inf")
MTILE = 256


def _gelu(x):
    return 0.5 * x * (1.0 + lax.erf(x * (2.0 ** -0.5)))


def _gate_body(E, tokens_ref, task_ids_ref, task_embed_ref, gate_w_ref,
               gate_b_ref,
               logits_ref, pos1_ref, pos2_ref, g1_ref, g2_ref,
               teid_ref, om_ref):
    N = tokens_ref.shape[1]
    D = tokens_ref.shape[2]
    x = tokens_ref[0]
    tid = task_ids_ref[0]
    te = task_embed_ref[...]
    tlog = jnp.dot(te, gate_w_ref[D:, :])
    tio = lax.broadcasted_iota(I32, tlog.shape, 0)
    tsel = jnp.sum(jnp.where(tio == tid, tlog, 0.0), axis=0, keepdims=True)
    logits = (jnp.dot(x, gate_w_ref[:D, :])
              + tsel + gate_b_ref[...][None, :])
    logits_ref[0] = logits
    io8 = lax.broadcasted_iota(I32, (N, E), 1)
    v1 = jnp.max(logits, axis=1, keepdims=True)
    i1 = jnp.min(jnp.where(logits == v1, io8, E), axis=1, keepdims=True)
    is1 = io8 == i1
    neg = jnp.where(is1, NEG_INF, logits)
    v2 = jnp.max(neg, axis=1, keepdims=True)
    i2 = jnp.min(jnp.where(neg == v2, io8, E), axis=1, keepdims=True)
    is2 = io8 == i2
    r = jnp.exp(v2 - v1)
    g1 = 1.0 / (1.0 + r)
    g1_ref[...] = g1
    g2_ref[...] = r * g1
    om_ref[...] = 1.0 - g1

    selF = jnp.where(is1 | is2, 1.0, 0.0)
    c = selF
    sh = 1
    while sh < N:
        shifted = jnp.concatenate(
            [jnp.zeros((sh, E), F32), c[:N - sh]], axis=0)
        c = c + shifted
        sh *= 2
    c_excl = c - selF
    total = c[N - 1:N, :]
    padded = jnp.floor((total + (MTILE - 1)) * (1.0 / MTILE)) * MTILE
    offinc = padded
    sh = 1
    while sh < E:
        shifted = jnp.concatenate(
            [jnp.zeros((1, sh), F32), offinc[:, :E - sh]], axis=1)
        offinc = offinc + shifted
        sh *= 2
    off = offinc - padded
    offend = offinc
    pos1 = jnp.sum(jnp.where(is1, off + c_excl, 0.0),
                   axis=1, keepdims=True)
    pos2 = jnp.sum(jnp.where(is2, off + c_excl, 0.0),
                   axis=1, keepdims=True)
    pos1_ref[...] = pos1.astype(I32)
    pos2_ref[...] = pos2.astype(I32)
    tb = (lax.broadcasted_iota(I32, (1, 32), 1) * MTILE).astype(F32)
    acc = jnp.zeros((1, 32), F32)
    for e2 in range(E):
        acc = acc + jnp.where(tb >= offend[:, e2:e2 + 1], 1.0, 0.0)
    teid_ref[...] = acc.astype(I32)


def _univ_body(tokens_ref, om_ref, uw1_ref, ub1_ref, uw2_ref, ub2_ref,
               tu_ref, xbf_s):
    f = pl.program_id(0)

    @pl.when(f == 0)
    def _cast():
        xbf_s[...] = tokens_ref[0].astype(BF16)

    xb = xbf_s[...]
    h = jnp.dot(xb, uw1_ref[...].astype(BF16), preferred_element_type=F32)
    h = _gelu(h + ub1_ref[0, 0][None, :])
    y = jnp.dot(h.astype(BF16), uw2_ref[...].astype(BF16),
                preferred_element_type=F32)
    y = jnp.where(f == 0, y + ub2_ref[0, 0][None, :], y)
    val = om_ref[...] * y

    @pl.when(f == 0)
    def _init():
        tu_ref[0] = val

    @pl.when(f != 0)
    def _add():
        tu_ref[0] += val


def _scatter_rows_body(N, P, pos1_hbm, pos2_hbm, g1_hbm, g2_hbm,
                       row_token_hbm, row_gate_hbm,
                       p1_v, p2_v, g1buf_v, g2buf_v, tok_v, zi_v, zf_v, sem):
    cid = lax.axis_index("c")
    sid = lax.axis_index("s")
    TOK = N // 16
    tok0 = sid * TOK
    io16 = lax.broadcasted_iota(I32, (16,), 0)
    ZC = P // 16

    @pl.when(cid == 0)
    def _zero():
        for k in range(ZC // 16):
            zi_v[pl.ds(k * 16, 16)] = jnp.zeros((16,), I32)
            zf_v[pl.ds(k * 16, 16)] = jnp.zeros((16,), F32)
        pltpu.sync_copy(zi_v, row_token_hbm.at[pl.ds(sid * ZC, ZC)])
        pltpu.sync_copy(zf_v, row_gate_hbm.at[pl.ds(sid * ZC, ZC)])

    plsc.subcore_barrier()

    @pl.when(cid == 0)
    def _scatter():
        pltpu.sync_copy(pos1_hbm.at[pl.ds(tok0, TOK)], p1_v)
        pltpu.sync_copy(pos2_hbm.at[pl.ds(tok0, TOK)], p2_v)
        pltpu.sync_copy(g1_hbm.at[pl.ds(tok0, TOK)], g1buf_v)
        pltpu.sync_copy(g2_hbm.at[pl.ds(tok0, TOK)], g2buf_v)
        for k in range(TOK // 16):
            tok_v[pl.ds(k * 16, 16)] = tok0 + k * 16 + io16
        pltpu.async_copy(tok_v, row_token_hbm.at[p1_v], sem).wait()
        pltpu.async_copy(g1buf_v, row_gate_hbm.at[p1_v], sem).wait()
        pltpu.async_copy(tok_v, row_token_hbm.at[p2_v], sem).wait()
        pltpu.async_copy(g2buf_v, row_gate_hbm.at[p2_v], sem).wait()


def _gather_body(P, D, row_token_hbm, tokens_hbm, xs_hbm, idx_v, rows_v, sem):
    cid = lax.axis_index("c")
    sid = lax.axis_index("s")
    wid = sid * 2 + cid
    ROWS = P // 32
    CH = 32
    for ch in range(ROWS // CH):
        p0 = wid * ROWS + ch * CH
        pltpu.sync_copy(row_token_hbm.at[pl.ds(p0, CH)], idx_v)
        pltpu.async_copy(tokens_hbm.at[idx_v], rows_v, sem).wait()
        pltpu.sync_copy(rows_v, xs_hbm.at[pl.ds(p0, CH)])


def _grouped_body(eid_sref, x_ref, gate_ref, w1_ref, b1_ref, w2_ref, b2_ref,
                  y_ref):
    t = pl.program_id(0)
    f = pl.program_id(1)
    eid = eid_sref[t]

    @pl.when(eid < 8)
    def _compute():
        xb = x_ref[...].astype(BF16)
        h = jnp.dot(xb, w1_ref[0].astype(BF16), preferred_element_type=F32)
        h = _gelu(h + b1_ref[0, 0][None, :])
        y = jnp.dot(h.astype(BF16), w2_ref[0].astype(BF16),
                    preferred_element_type=F32)
        y = jnp.where(f == 0, y + b2_ref[0, 0][None, :], y)
        val = gate_ref[...] * y

        @pl.when(f == 0)
        def _init():
            y_ref[...] = val

        @pl.when(f != 0)
        def _add():
            y_ref[...] += val


def _combine_body(N, D, y_hbm, tu_hbm, pos1_hbm, pos2_hbm, out_hbm,
                  i1_v, i2_v, b1_v, b2_v, b3_v, sem):
    cid = lax.axis_index("c")
    sid = lax.axis_index("s")
    wid = sid * 2 + cid
    TOK = N // 32
    CH = 32
    for ch in range(TOK // CH):
        t0 = wid * TOK + ch * CH
        pltpu.sync_copy(pos1_hbm.at[pl.ds(t0, CH)], i1_v)
        pltpu.sync_copy(pos2_hbm.at[pl.ds(t0, CH)], i2_v)
        pltpu.async_copy(y_hbm.at[i1_v], b1_v, sem).wait()
        pltpu.async_copy(y_hbm.at[i2_v], b2_v, sem).wait()
        pltpu.sync_copy(tu_hbm.at[pl.ds(t0, CH)], b3_v)

        def _row(r, carry):
            for cc in range(D // 16):
                sl = pl.ds(cc * 16, 16)
                b3_v[r, sl] = b3_v[r, sl] + b1_v[r, sl] + b2_v[r, sl]
            return carry

        lax.fori_loop(0, CH, _row, 0)
        pltpu.sync_copy(b3_v, out_hbm.at[pl.ds(t0, CH)])


def _moe_sparse_core(tokens, task_ids, task_embed, gate_w, gate_b,
                     w1, b1, w2, b2, uw1, ub1, uw2, ub2):
    B, N, D = tokens.shape
    E = gate_w.shape[1]
    T = task_embed.shape[0]
    F = w1.shape[2]
    FC = 1024
    NF = F // FC
    NTILES = (2 * N) // MTILE + E
    P = NTILES * MTILE

    body1 = functools.partial(_gate_body, E)
    logits, pos1, pos2, g1, g2, teid, om = pl.pallas_call(
        body1,
        grid=(1,),
        in_specs=[
            pl.BlockSpec((1, N, D), lambda f: (0, 0, 0)),
            pl.BlockSpec(memory_space=pltpu.SMEM),
            pl.BlockSpec((T, D), lambda f: (0, 0)),
            pl.BlockSpec((2 * D, E), lambda f: (0, 0)),
            pl.BlockSpec((E,), lambda f: (0,)),
        ],
        out_specs=[
            pl.BlockSpec((1, N, E), lambda f: (0, 0, 0)),
            pl.BlockSpec((N, 1), lambda f: (0, 0)),
            pl.BlockSpec((N, 1), lambda f: (0, 0)),
            pl.BlockSpec((N, 1), lambda f: (0, 0)),
            pl.BlockSpec((N, 1), lambda f: (0, 0)),
            pl.BlockSpec((1, 32), lambda f: (0, 0)),
            pl.BlockSpec((N, 1), lambda f: (0, 0)),
        ],
        out_shape=[
            jax.ShapeDtypeStruct((B, N, E), F32),
            jax.ShapeDtypeStruct((N, 1), I32),
            jax.ShapeDtypeStruct((N, 1), I32),
            jax.ShapeDtypeStruct((N, 1), F32),
            jax.ShapeDtypeStruct((N, 1), F32),
            jax.ShapeDtypeStruct((1, 32), I32),
            jax.ShapeDtypeStruct((N, 1), F32),
        ],
    )(tokens, task_ids, task_embed, gate_w, gate_b)

    mesh = plsc.VectorSubcoreMesh(core_axis_name="c", subcore_axis_name="s")
    TOK = N // 16
    tile_eid = teid.reshape(32)

    scat = functools.partial(
        pl.kernel,
        mesh=mesh,
        out_type=[
            jax.ShapeDtypeStruct((P,), I32),
            jax.ShapeDtypeStruct((P,), F32),
        ],
        scratch_types=[
            pltpu.VMEM((TOK,), I32),
            pltpu.VMEM((TOK,), I32),
            pltpu.VMEM((TOK,), F32),
            pltpu.VMEM((TOK,), F32),
            pltpu.VMEM((TOK,), I32),
            pltpu.VMEM((P // 16,), I32),
            pltpu.VMEM((P // 16,), F32),
            pltpu.SemaphoreType.DMA,
        ],
    )(functools.partial(_scatter_rows_body, N, P))
    row_token, row_gate = scat(
        pos1.reshape(N), pos2.reshape(N), g1.reshape(N), g2.reshape(N))

    gath = functools.partial(
        pl.kernel,
        mesh=mesh,
        out_type=[jax.ShapeDtypeStruct((P, D), F32)],
        scratch_types=[
            pltpu.VMEM((32,), I32),
            pltpu.VMEM((32, D), F32),
            pltpu.SemaphoreType.DMA,
        ],
    )(functools.partial(_gather_body, P, D))
    (x_sorted,) = gath(row_token, tokens.reshape(N, D))

    tu = pl.pallas_call(
        _univ_body,
        grid=(NF,),
        in_specs=[
            pl.BlockSpec((1, N, D), lambda f: (0, 0, 0)),
            pl.BlockSpec((N, 1), lambda f: (0, 0)),
            pl.BlockSpec((D, FC), lambda f: (0, f)),
            pl.BlockSpec((1, 1, FC), lambda f: (0, 0, f)),
            pl.BlockSpec((FC, D), lambda f: (f, 0)),
            pl.BlockSpec((1, 1, D), lambda f: (0, 0, 0)),
        ],
        out_specs=pl.BlockSpec((1, N, D), lambda f: (0, 0, 0)),
        out_shape=jax.ShapeDtypeStruct((B, N, D), F32),
        scratch_shapes=[pltpu.VMEM((N, D), BF16)],
    )(tokens, om, uw1, ub1.reshape(1, 1, F), uw2, ub2.reshape(1, 1, D))

    grid_spec = pltpu.PrefetchScalarGridSpec(
        num_scalar_prefetch=1,
        grid=(NTILES, NF),
        in_specs=[
            pl.BlockSpec((MTILE, D), lambda t, f, eid: (t, 0)),
            pl.BlockSpec((MTILE, 1), lambda t, f, eid: (t, 0)),
            pl.BlockSpec((1, D, FC), lambda t, f, eid: (jnp.minimum(eid[t], 7), 0, f)),
            pl.BlockSpec((1, 1, FC), lambda t, f, eid: (jnp.minimum(eid[t], 7), 0, f)),
            pl.BlockSpec((1, FC, D), lambda t, f, eid: (jnp.minimum(eid[t], 7), f, 0)),
            pl.BlockSpec((1, 1, D), lambda t, f, eid: (jnp.minimum(eid[t], 7), 0, 0)),
        ],
        out_specs=pl.BlockSpec((MTILE, D), lambda t, f, eid: (t, 0)),
    )
    y = pl.pallas_call(
        _grouped_body,
        grid_spec=grid_spec,
        out_shape=jax.ShapeDtypeStruct((P, D), F32),
    )(tile_eid, x_sorted, row_gate.reshape(P, 1),
      w1, b1.reshape(E, 1, F), w2, b2.reshape(E, 1, D))

    comb = functools.partial(
        pl.kernel,
        mesh=mesh,
        out_type=[jax.ShapeDtypeStruct((N, D), F32)],
        scratch_types=[
            pltpu.VMEM((32,), I32),
            pltpu.VMEM((32,), I32),
            pltpu.VMEM((32, D), F32),
            pltpu.VMEM((32, D), F32),
            pltpu.VMEM((32, D), F32),
            pltpu.SemaphoreType.DMA,
        ],
    )(functools.partial(_combine_body, N, D))
    (t_out,) = comb(y, tu.reshape(N, D), pos1.reshape(N), pos2.reshape(N))

    return (t_out.reshape(B, N, D), logits, pos1, pos2, g1, g2, tile_eid,
            row_token, row_gate, x_sorted, y, tu)


def kernel(tokens, task_ids, task_embed, gate_w, gate_b,
           w1, b1, w2, b2, uw1, ub1, uw2, ub2):
    t_out, logits = _moe_sparse_core(
        tokens, task_ids, task_embed, gate_w, gate_b,
        w1, b1, w2, b2, uw1, ub1, uw2, ub2)[:2]
    return t_out, logits
